# baseline (speedup 1.0000x reference)
import functools

import jax
import jax.numpy as jnp
from jax.experimental import pallas as pl
from jax.experimental.pallas import tpu as pltpu

BATCH = 4096
TILE = 1024
HALF = 512
INPUT_DIM = 768
EMBED_DIM = 64
CODEBOOK_SIZE = 1024
N_LAYERS = 3
N_CAT = 18
COMMIT_W = 0.25

_W_SHAPES = [(INPUT_DIM, 2048), (2048, 1024), (1024, 512), (512, EMBED_DIM),
             (EMBED_DIM, 512), (512, 1024), (1024, 2048), (2048, INPUT_DIM)]


def _silu(v):
    return 0.5 * v * (1.0 + jnp.tanh(0.5 * v))


def _dot(a, b):
    return jax.lax.dot_general(a, b, (((1,), (0,)), ((), ())),
                               preferred_element_type=jnp.float32)


def _encode(xb, ws, bs, first, hbm, sems):
    h = xb
    for li in range(4):
        @pl.when(first)
        def _wait():
            pltpu.make_async_copy(hbm[li], ws[li], sems[li]).wait()
        h = _dot(h, ws[li][...]) + bs[li]
        if li != 3:
            h = _silu(h)
    n = jnp.sqrt(jnp.sum(h * h, axis=-1, keepdims=True))
    return h / jnp.maximum(n, 1e-12)


def _quantize3(res, cbs):
    emb_sum = jnp.zeros((HALF, EMBED_DIM), dtype=jnp.float32)
    rq_row = jnp.zeros((HALF, 1), dtype=jnp.float32)
    for cb, half_cb_sq in cbs:
        t = _dot(res, cb.T) - half_cb_sq
        tmax = jnp.max(t, axis=-1, keepdims=True)
        onehot = (t == tmax).astype(jnp.float32)
        emb = _dot(onehot, cb)
        rn = jnp.sqrt(jnp.sum(res * res, axis=-1, keepdims=True))
        u = res / (rn + 1e-8)
        qn = jnp.sqrt(jnp.sum(emb * emb, axis=-1, keepdims=True))
        q = emb / (qn + 1e-8)
        w = u + q
        wn = jnp.sqrt(jnp.sum(w * w, axis=-1, keepdims=True))
        w = w / jnp.maximum(wn, 1e-6)
        ew = jnp.sum(res * w, axis=-1, keepdims=True)
        eu = jnp.sum(res * u, axis=-1, keepdims=True)
        out = res - 2.0 * ew * w + 2.0 * eu * q
        new_res = res - out
        rq_row = rq_row + jnp.sum(new_res * new_res, axis=-1, keepdims=True)
        emb_sum = emb_sum + out
        res = new_res
    return emb_sum, rq_row


def _decode_losses(xb, emb_sum, rq_row, ws, bs, first, hbm, sems):
    h = emb_sum
    for li in range(4):
        @pl.when(first)
        def _wait():
            pltpu.make_async_copy(hbm[li], ws[li], sems[li]).wait()
        h = _dot(h, ws[li][...]) + bs[li]
        if li != 3:
            h = _silu(h)
    x_hat = h

    colx = jax.lax.broadcasted_iota(jnp.int32, (HALF, INPUT_DIM), 1)
    is_mse = colx < (INPUT_DIM - N_CAT)
    diff = x_hat - xb
    mse_row = jnp.sum(jnp.where(is_mse, diff * diff, 0.0), axis=-1)
    lg = x_hat[:, INPUT_DIM - 128:]
    tg = xb[:, INPUT_DIM - 128:]
    colb = jax.lax.broadcasted_iota(jnp.int32, (HALF, 128), 1)
    bce = (jnp.maximum(lg, 0.0) - lg * tg
           + jnp.log1p(jnp.exp(-jnp.abs(lg))))
    bce_row = jnp.sum(jnp.where(colb >= 128 - N_CAT, bce, 0.0), axis=-1)
    recon_sum = jnp.sum(mse_row + bce_row)
    rq_sum = (1.0 + COMMIT_W) * jnp.sum(rq_row)
    return recon_sum, rq_sum


def _fused_body(x_ref,
                ew0h, ew1h, ew2h, ew3h, dw0h, dw1h, dw2h, dw3h,
                eb0, eb1, eb2, eb3, db0, db1, db2, db3,
                cb0, cb1, cb2,
                recon_ref, rq_ref,
                ew0, ew1, ew2, ew3, dw0, dw1, dw2, dw3,
                s0, s1, s2, s3, s4, s5, s6, s7):
    i = pl.program_id(0)
    first = i == 0
    enc_hbm = (ew0h, ew1h, ew2h, ew3h)
    dec_hbm = (dw0h, dw1h, dw2h, dw3h)
    enc_w = (ew0, ew1, ew2, ew3)
    dec_w = (dw0, dw1, dw2, dw3)
    enc_s = (s0, s1, s2, s3)
    dec_s = (s4, s5, s6, s7)

    @pl.when(first)
    def _start_dmas():
        for hbm, vmem, sem in zip(enc_hbm + dec_hbm, enc_w + dec_w,
                                  enc_s + dec_s):
            pltpu.make_async_copy(hbm, vmem, sem).start()

    enc_b = (eb0[...], eb1[...], eb2[...], eb3[...])
    dec_b = (db0[...], db1[...], db2[...], db3[...])
    cbs = []
    for cb_ref in (cb0, cb1, cb2):
        cbm = cb_ref[...]
        cbn = jnp.sqrt(jnp.sum(cbm * cbm, axis=-1, keepdims=True))
        cb = cbm / jnp.maximum(cbn, 1e-12)
        cbs.append((cb, 0.5 * jnp.sum(cb * cb, axis=-1)[None, :]))

    xb0 = x_ref[0:HALF, :]
    xb1 = x_ref[HALF:TILE, :]
    res0 = _encode(xb0, enc_w, enc_b, first, enc_hbm, enc_s)
    res1 = _encode(xb1, enc_w, enc_b, i < 0, enc_hbm, enc_s)
    emb0, rqr0 = _quantize3(res0, cbs)
    emb1, rqr1 = _quantize3(res1, cbs)
    recon0, rq0 = _decode_losses(xb0, emb0, rqr0, dec_w, dec_b, first,
                                 dec_hbm, dec_s)
    recon1, rq1 = _decode_losses(xb1, emb1, rqr1, dec_w, dec_b, i < 0,
                                 dec_hbm, dec_s)
    recon_sum = recon0 + recon1
    rq_sum = rq0 + rq1

    @pl.when(first)
    def _init():
        recon_ref[...] = jnp.zeros((1, 1), jnp.float32)
        rq_ref[...] = jnp.zeros((1, 1), jnp.float32)

    recon_ref[...] += recon_sum.reshape(1, 1)
    rq_ref[...] += rq_sum.reshape(1, 1)


@functools.partial(jax.jit, static_argnames=())
def _fused(x, ew0, ew1, ew2, ew3, dw0, dw1, dw2, dw3,
           eb0, eb1, eb2, eb3, db0, db1, db2, db3, cb0, cb1, cb2):
    num_tiles = BATCH // TILE

    def vspec(shape):
        return pl.BlockSpec(shape, lambda i: (0,) * len(shape))

    in_specs = [pl.BlockSpec((TILE, INPUT_DIM), lambda i: (i, 0))]
    in_specs += [pl.BlockSpec(memory_space=pl.ANY)] * 8
    small = [eb0, eb1, eb2, eb3, db0, db1, db2, db3, cb0, cb1, cb2]
    in_specs += [vspec(a.shape) for a in small]

    out_shape = (jax.ShapeDtypeStruct((1, 1), jnp.float32),
                 jax.ShapeDtypeStruct((1, 1), jnp.float32))
    out_specs = (pl.BlockSpec((1, 1), lambda i: (0, 0)),
                 pl.BlockSpec((1, 1), lambda i: (0, 0)))

    scratch_shapes = ([pltpu.VMEM(s, jnp.float32) for s in _W_SHAPES]
                      + [pltpu.SemaphoreType.DMA] * 8)

    recon_sum, rq_sum = pl.pallas_call(
        _fused_body,
        grid=(num_tiles,),
        in_specs=in_specs,
        out_specs=out_specs,
        out_shape=out_shape,
        scratch_shapes=scratch_shapes,
        compiler_params=pltpu.CompilerParams(
            dimension_semantics=("arbitrary",),
            vmem_limit_bytes=128 * 1024 * 1024,
        ),
    )(x, ew0, ew1, ew2, ew3, dw0, dw1, dw2, dw3, *small)
    return recon_sum[0, 0], rq_sum[0, 0]


def kernel(x, enc_w0, enc_b0, enc_w1, enc_b1, enc_w2, enc_b2, enc_w3, enc_b3,
           dec_w0, dec_b0, dec_w1, dec_b1, dec_w2, dec_b2, dec_w3, dec_b3,
           cb0, cb1, cb2, gumbel_t):
    del gumbel_t
    bs = [b.reshape(1, -1) for b in
          (enc_b0, enc_b1, enc_b2, enc_b3, dec_b0, dec_b1, dec_b2, dec_b3)]
    recon_sum, rq_sum = _fused(
        x, enc_w0, enc_w1, enc_w2, enc_w3, dec_w0, dec_w1, dec_w2, dec_w3,
        *bs, cb0, cb1, cb2)
    recon_mean = recon_sum / BATCH
    rq_mean = rq_sum / BATCH
    loss = recon_mean + 3.0 * rq_mean
    return loss, recon_mean, rq_mean


# speedup vs baseline: 1.1815x; 1.1815x over previous
import functools

import jax
import jax.numpy as jnp
from jax.experimental import pallas as pl
from jax.experimental.pallas import tpu as pltpu

BATCH = 4096
TILE = 1024
HALF = 512
INPUT_DIM = 768
EMBED_DIM = 64
CODEBOOK_SIZE = 1024
N_LAYERS = 3
N_CAT = 18
COMMIT_W = 0.25

_DW_SHAPES = [(EMBED_DIM, 512), (512, 1024), (1024, 2048), (2048, INPUT_DIM)]


def _silu(v):
    return 0.5 * v * (1.0 + jnp.tanh(0.5 * v))


def _dot(a, b):
    return jax.lax.dot_general(a, b, (((1,), (0,)), ((), ())),
                               preferred_element_type=jnp.float32)


def _encode(xb, enc):
    (ew0, eb0, ew1, eb1, ew2, eb2, ew3, eb3) = enc
    h = _silu(_dot(xb, ew0) + eb0)
    h = _silu(_dot(h, ew1) + eb1)
    h = _silu(_dot(h, ew2) + eb2)
    h = _dot(h, ew3) + eb3
    n = jnp.sqrt(jnp.sum(h * h, axis=-1, keepdims=True))
    return h / jnp.maximum(n, 1e-12)


def _quantize3(res, cbs):
    emb_sum = jnp.zeros((HALF, EMBED_DIM), dtype=jnp.float32)
    rq_row = jnp.zeros((HALF, 1), dtype=jnp.float32)
    for cb, half_cb_sq in cbs:
        t = _dot(res, cb.T) - half_cb_sq
        tmax = jnp.max(t, axis=-1, keepdims=True)
        onehot = (t == tmax).astype(jnp.float32)
        emb = _dot(onehot, cb)
        rn = jnp.sqrt(jnp.sum(res * res, axis=-1, keepdims=True))
        u = res / (rn + 1e-8)
        qn = jnp.sqrt(jnp.sum(emb * emb, axis=-1, keepdims=True))
        q = emb / (qn + 1e-8)
        w = u + q
        wn = jnp.sqrt(jnp.sum(w * w, axis=-1, keepdims=True))
        w = w / jnp.maximum(wn, 1e-6)
        ew = jnp.sum(res * w, axis=-1, keepdims=True)
        eu = jnp.sum(res * u, axis=-1, keepdims=True)
        out = res - 2.0 * ew * w + 2.0 * eu * q
        new_res = res - out
        rq_row = rq_row + jnp.sum(new_res * new_res, axis=-1, keepdims=True)
        emb_sum = emb_sum + out
        res = new_res
    return emb_sum, rq_row


def _decode_losses(xb, emb_sum, rq_row, dec):
    (dw0, db0, dw1, db1, dw2, db2, dw3, db3) = dec
    h = _silu(_dot(emb_sum, dw0) + db0)
    h = _silu(_dot(h, dw1) + db1)
    h = _silu(_dot(h, dw2) + db2)
    x_hat = _dot(h, dw3) + db3

    colx = jax.lax.broadcasted_iota(jnp.int32, (HALF, INPUT_DIM), 1)
    is_mse = colx < (INPUT_DIM - N_CAT)
    diff = x_hat - xb
    mse_row = jnp.sum(jnp.where(is_mse, diff * diff, 0.0), axis=-1)
    lg = x_hat[:, INPUT_DIM - 128:]
    tg = xb[:, INPUT_DIM - 128:]
    colb = jax.lax.broadcasted_iota(jnp.int32, (HALF, 128), 1)
    bce = (jnp.maximum(lg, 0.0) - lg * tg
           + jnp.log1p(jnp.exp(-jnp.abs(lg))))
    bce_row = jnp.sum(jnp.where(colb >= 128 - N_CAT, bce, 0.0), axis=-1)
    recon_sum = jnp.sum(mse_row + bce_row)
    rq_sum = (1.0 + COMMIT_W) * jnp.sum(rq_row)
    return recon_sum, rq_sum


def _fused_body(x_ref,
                ew0, eb0, ew1, eb1, ew2, eb2, ew3, eb3,
                dw0h, dw1h, dw2h, dw3h,
                db0, db1, db2, db3,
                cb0, cb1, cb2,
                recon_ref, rq_ref,
                dw0, dw1, dw2, dw3,
                s0, s1, s2, s3):
    i = pl.program_id(0)
    first = i == 0
    dec_hbm = (dw0h, dw1h, dw2h, dw3h)
    dec_vw = (dw0, dw1, dw2, dw3)
    dec_s = (s0, s1, s2, s3)

    @pl.when(first)
    def _start_dmas():
        for hbm, vmem, sem in zip(dec_hbm, dec_vw, dec_s):
            pltpu.make_async_copy(hbm, vmem, sem).start()

    enc = (ew0[...], eb0[...], ew1[...], eb1[...],
           ew2[...], eb2[...], ew3[...], eb3[...])
    cbs = []
    for cb_ref in (cb0, cb1, cb2):
        cbm = cb_ref[...]
        cbn = jnp.sqrt(jnp.sum(cbm * cbm, axis=-1, keepdims=True))
        cb = cbm / jnp.maximum(cbn, 1e-12)
        cbs.append((cb, 0.5 * jnp.sum(cb * cb, axis=-1)[None, :]))

    xb0 = x_ref[0:HALF, :]
    xb1 = x_ref[HALF:TILE, :]
    res0 = _encode(xb0, enc)
    res1 = _encode(xb1, enc)
    emb0, rqr0 = _quantize3(res0, cbs)
    emb1, rqr1 = _quantize3(res1, cbs)

    @pl.when(first)
    def _wait_dec():
        for hbm, vmem, sem in zip(dec_hbm, dec_vw, dec_s):
            pltpu.make_async_copy(hbm, vmem, sem).wait()

    dec = (dw0[...], db0[...], dw1[...], db1[...],
           dw2[...], db2[...], dw3[...], db3[...])
    recon0, rq0 = _decode_losses(xb0, emb0, rqr0, dec)
    recon1, rq1 = _decode_losses(xb1, emb1, rqr1, dec)
    recon_sum = recon0 + recon1
    rq_sum = rq0 + rq1

    @pl.when(first)
    def _init():
        recon_ref[...] = jnp.zeros((1, 1), jnp.float32)
        rq_ref[...] = jnp.zeros((1, 1), jnp.float32)

    recon_ref[...] += recon_sum.reshape(1, 1)
    rq_ref[...] += rq_sum.reshape(1, 1)


@functools.partial(jax.jit, static_argnames=())
def _fused(x, ew0, eb0, ew1, eb1, ew2, eb2, ew3, eb3,
           dw0, dw1, dw2, dw3, db0, db1, db2, db3, cb0, cb1, cb2):
    num_tiles = BATCH // TILE

    def vspec(a):
        shape = a.shape
        return pl.BlockSpec(shape, lambda i: (0,) * len(shape))

    enc_in = [ew0, eb0, ew1, eb1, ew2, eb2, ew3, eb3]
    small = [db0, db1, db2, db3, cb0, cb1, cb2]
    in_specs = ([pl.BlockSpec((TILE, INPUT_DIM), lambda i: (i, 0))]
                + [vspec(a) for a in enc_in]
                + [pl.BlockSpec(memory_space=pl.ANY)] * 4
                + [vspec(a) for a in small])

    out_shape = (jax.ShapeDtypeStruct((1, 1), jnp.float32),
                 jax.ShapeDtypeStruct((1, 1), jnp.float32))
    out_specs = (pl.BlockSpec((1, 1), lambda i: (0, 0)),
                 pl.BlockSpec((1, 1), lambda i: (0, 0)))

    scratch_shapes = ([pltpu.VMEM(s, jnp.float32) for s in _DW_SHAPES]
                      + [pltpu.SemaphoreType.DMA] * 4)

    recon_sum, rq_sum = pl.pallas_call(
        _fused_body,
        grid=(num_tiles,),
        in_specs=in_specs,
        out_specs=out_specs,
        out_shape=out_shape,
        scratch_shapes=scratch_shapes,
        compiler_params=pltpu.CompilerParams(
            dimension_semantics=("arbitrary",),
            vmem_limit_bytes=128 * 1024 * 1024,
        ),
    )(x, *enc_in, dw0, dw1, dw2, dw3, *small)
    return recon_sum[0, 0], rq_sum[0, 0]


def kernel(x, enc_w0, enc_b0, enc_w1, enc_b1, enc_w2, enc_b2, enc_w3, enc_b3,
           dec_w0, dec_b0, dec_w1, dec_b1, dec_w2, dec_b2, dec_w3, dec_b3,
           cb0, cb1, cb2, gumbel_t):
    del gumbel_t
    eb = [b.reshape(1, -1) for b in (enc_b0, enc_b1, enc_b2, enc_b3)]
    db = [b.reshape(1, -1) for b in (dec_b0, dec_b1, dec_b2, dec_b3)]
    recon_sum, rq_sum = _fused(
        x, enc_w0, eb[0], enc_w1, eb[1], enc_w2, eb[2], enc_w3, eb[3],
        dec_w0, dec_w1, dec_w2, dec_w3, db[0], db[1], db[2], db[3],
        cb0, cb1, cb2)
    recon_mean = recon_sum / BATCH
    rq_mean = rq_sum / BATCH
    loss = recon_mean + 3.0 * rq_mean
    return loss, recon_mean, rq_mean
